# baseline (speedup 1.0000x reference)
import functools

import jax
import jax.numpy as jnp
from jax import lax
from jax.experimental import pallas as pl
from jax.experimental.pallas import tpu as pltpu
from jax.experimental.pallas import tpu_sc as plsc

_B = 1024
_C = 100000
_K = 10

_NC = 2
_NS = 16
_NW = _NC * _NS
_L = 16
_RW = 128
_BPW = _B // _NW

_SC_MESH = dict(core_axis_name="c", subcore_axis_name="s")


def _sc_gather_body(logits_hbm, labels_hbm, out_hbm, lab_v, row_v, rows_v,
                    val_v, sem):
    wid = lax.axis_index("s") * _NC + lax.axis_index("c")
    base = wid * _BPW
    pltpu.sync_copy(labels_hbm.at[pl.ds(base, _BPW)], lab_v)
    for g in range(_BPW // _L):
        lab = lab_v[pl.ds(g * _L, _L)]
        bidx = lax.iota(jnp.int32, _L) + (base + g * _L)
        flat = bidx * _C + lab
        row_v[pl.ds(g * _L, _L)] = lax.shift_right_logical(flat, 7)
    pltpu.async_copy(logits_hbm.at[row_v], rows_v, sem).wait()
    for g in range(_BPW // _L):
        lab = lab_v[pl.ds(g * _L, _L)]
        bidx = lax.iota(jnp.int32, _L) + (base + g * _L)
        lane = lax.bitwise_and(bidx * _C + lab, jnp.int32(_RW - 1))
        rowi = lax.iota(jnp.int32, _L) + g * _L
        val_v[pl.ds(g * _L, _L)] = plsc.load_gather(rows_v, [rowi, lane])
    pltpu.sync_copy(val_v, out_hbm.at[pl.ds(base, _BPW)])


def _sc_gather(logits, labels):
    k = functools.partial(
        pl.kernel,
        mesh=plsc.VectorSubcoreMesh(**_SC_MESH),
        compiler_params=pltpu.CompilerParams(needs_layout_passes=False),
        out_type=jax.ShapeDtypeStruct((_B,), jnp.float32),
        scratch_types=[
            pltpu.VMEM((_BPW,), jnp.int32),
            pltpu.VMEM((_BPW,), jnp.int32),
            pltpu.VMEM((_BPW, _RW), jnp.float32),
            pltpu.VMEM((_BPW,), jnp.float32),
            pltpu.SemaphoreType.DMA,
        ],
    )(_sc_gather_body)
    return k(logits.reshape(_B * _C // _RW, _RW), labels)


_LANES = 128
_PREFIX = 2048


def _slice_update(acc_ref, x_ref, vt, labm, c, chunk, s, extra_mask=None):
    xs = x_ref[:, s * _LANES:(s + 1) * _LANES]
    base = c * chunk + s * _LANES
    m_ge = xs >= vt
    m_gt = xs > vt
    if extra_mask is not None:
        m_ge = m_ge & extra_mask
        m_gt = m_gt & extra_mask
    mc = labm > base
    f_ge = jnp.where(m_ge, 1.0, 0.0)
    f_gt = jnp.where(m_gt, 1.0, 0.0)
    acc_ref[...] += jnp.where(mc, f_ge, f_gt)


def _prefix_body(v_ref, labm_ref, x_ref, out_ref, acc_ref):
    acc_ref[...] = jnp.zeros_like(acc_ref)
    vt = v_ref[...]
    labm = labm_ref[...]
    for s in range(_PREFIX // _LANES):
        _slice_update(acc_ref, x_ref, vt, labm, 0, _PREFIX, s)
    out_ref[...] = jnp.sum(acc_ref[...], axis=1, keepdims=True)


def _tc_prefix(logits, vb, labm):
    return pl.pallas_call(
        _prefix_body,
        grid=(1,),
        in_specs=[
            pl.BlockSpec((_B, _LANES), lambda c: (0, 0)),
            pl.BlockSpec((_B, _LANES), lambda c: (0, 0)),
            pl.BlockSpec((_B, _PREFIX), lambda c: (0, 0)),
        ],
        out_specs=pl.BlockSpec((_B, 1), lambda c: (0, 0)),
        out_shape=jax.ShapeDtypeStruct((_B, 1), jnp.float32),
        scratch_shapes=[pltpu.VMEM((_B, _LANES), jnp.float32)],
    )(vb, labm, logits)


_CHUNK = 4096
_NCH = -(-_C // _CHUNK)
_TAIL = _C - (_NCH - 1) * _CHUNK
_TAIL_FULL = _TAIL // _LANES
_TAIL_REM = _TAIL - _TAIL_FULL * _LANES


def _full_body(v_ref, labm_ref, x_ref, out_ref, acc_ref):
    c = pl.program_id(0)

    @pl.when(c == 0)
    def _init():
        acc_ref[...] = jnp.zeros_like(acc_ref)

    vt = v_ref[...]
    labm = labm_ref[...]

    @pl.when(c < _NCH - 1)
    def _full():
        for s in range(_CHUNK // _LANES):
            _slice_update(acc_ref, x_ref, vt, labm, c, _CHUNK, s)

    @pl.when(c == _NCH - 1)
    def _tail_and_fini():
        for s in range(_TAIL_FULL):
            _slice_update(acc_ref, x_ref, vt, labm, c, _CHUNK, s)
        rem = lax.broadcasted_iota(jnp.int32, (_B, _LANES), 1) < _TAIL_REM
        _slice_update(acc_ref, x_ref, vt, labm, c, _CHUNK, _TAIL_FULL,
                      extra_mask=rem)
        counts = jnp.sum(acc_ref[...], axis=1, keepdims=True)
        correct = (counts < float(_K)).astype(jnp.float32)
        total = jnp.sum(correct) * (1.0 / _B)
        out_ref[...] = jnp.broadcast_to(total, (1, 1))


def _tc_full(logits, vb, labm):
    return pl.pallas_call(
        _full_body,
        grid=(_NCH,),
        in_specs=[
            pl.BlockSpec((_B, _LANES), lambda c: (0, 0)),
            pl.BlockSpec((_B, _LANES), lambda c: (0, 0)),
            pl.BlockSpec((_B, _CHUNK), lambda c: (0, c)),
        ],
        out_specs=pl.BlockSpec((1, 1), lambda c: (0, 0)),
        out_shape=jax.ShapeDtypeStruct((1, 1), jnp.float32),
        scratch_shapes=[pltpu.VMEM((_B, _LANES), jnp.float32)],
    )(vb, labm, logits).reshape(())


_GRP = 10
_NGRP = _C // _L
assert _NGRP % _GRP == 0


def _sc_rows_body(flat_hbm, rows_hbm, vu_hbm, labml_hbm, out_hbm,
                  meta_v, vu_v, labml_v, buf, val_v):
    wid = lax.axis_index("s") * _NC + lax.axis_index("c")
    pltpu.sync_copy(rows_hbm.at[wid], meta_v)
    pltpu.sync_copy(vu_hbm.at[wid], vu_v)
    pltpu.sync_copy(labml_hbm.at[wid], labml_v)
    row = jnp.max(meta_v[...])
    start = row * _C
    pltpu.sync_copy(flat_hbm.at[pl.ds(start, _C)], buf)
    vu = vu_v[...]
    labml = labml_v[...]

    def step(i, acc):
        for u in range(_GRP):
            g = i * _GRP + u
            xs = buf[pl.ds(g * _L, _L)]
            m_ge = xs >= vu
            m_gt = xs > vu
            mc = labml > g * _L
            f_ge = jnp.where(m_ge, 1.0, 0.0)
            f_gt = jnp.where(m_gt, 1.0, 0.0)
            acc = acc + jnp.where(mc, f_ge, f_gt)
        return acc

    acc = lax.fori_loop(0, _NGRP // _GRP, step, jnp.zeros((_L,), jnp.float32))
    count = jnp.sum(acc)
    val_v[...] = jnp.broadcast_to(count, (_L,))
    pltpu.sync_copy(val_v, out_hbm.at[wid])


def _sc_rows(flat_logits, rows32, vu, labu):
    meta = jnp.broadcast_to(rows32[:, None], (_NW, _L)).astype(jnp.int32)
    vu16 = jnp.broadcast_to(vu[:, None], (_NW, _L))
    labml = labu[:, None].astype(jnp.int32) - lax.broadcasted_iota(
        jnp.int32, (_NW, _L), 1)
    k = functools.partial(
        pl.kernel,
        mesh=plsc.VectorSubcoreMesh(**_SC_MESH),
        compiler_params=pltpu.CompilerParams(needs_layout_passes=False),
        out_type=jax.ShapeDtypeStruct((_NW, _L), jnp.float32),
        scratch_types=[
            pltpu.VMEM((_L,), jnp.int32),
            pltpu.VMEM((_L,), jnp.float32),
            pltpu.VMEM((_L,), jnp.int32),
            pltpu.VMEM((_C,), jnp.float32),
            pltpu.VMEM((_L,), jnp.float32),
        ],
    )(_sc_rows_body)
    return k(flat_logits, meta, vu16, labml)


def kernel(logits, labels):
    labels = labels.astype(jnp.int32)
    v = _sc_gather(logits, labels)

    vb = jnp.broadcast_to(v.reshape(_B, 1), (_B, _LANES))
    labm = labels.reshape(_B, 1) - lax.broadcasted_iota(
        jnp.int32, (_B, _LANES), 1)

    prefix = _tc_prefix(logits, vb, labm).reshape(_B)
    undecided = prefix < float(_K)
    n = jnp.sum(undecided.astype(jnp.int32))
    (rows32,) = jnp.nonzero(undecided, size=_NW, fill_value=0)
    rows32 = rows32.astype(jnp.int32)

    def few_undecided(logits, labels, v, rows32, n, vb, labm):
        vu = v[rows32]
        labu = labels[rows32]
        counts = _sc_rows(logits.reshape(-1), rows32, vu, labu)[:, 0]
        live = lax.broadcasted_iota(jnp.int32, (_NW,), 0) < n
        correct = (counts < float(_K)) & live
        return jnp.sum(correct.astype(jnp.float32)) * (1.0 / _B)

    def many_undecided(logits, labels, v, rows32, n, vb, labm):
        return _tc_full(logits, vb, labm)

    return lax.cond(n <= _NW, few_undecided, many_undecided,
                    logits, labels, v, rows32, n, vb, labm)


# speedup vs baseline: 1.2198x; 1.2198x over previous
import functools

import jax
import jax.numpy as jnp
from jax import lax
from jax.experimental import pallas as pl
from jax.experimental.pallas import tpu as pltpu
from jax.experimental.pallas import tpu_sc as plsc

_B = 1024
_C = 100000
_K = 10

_NC = 2
_NS = 16
_NW = _NC * _NS
_L = 16
_RW = 128
_BPW = _B // _NW

_SC_MESH = dict(core_axis_name="c", subcore_axis_name="s")


def _sc_gather_body(logits_hbm, labels_hbm, out_hbm, lab_v, row_v, rows_v,
                    val_v, sem):
    wid = lax.axis_index("s") * _NC + lax.axis_index("c")
    base = wid * _BPW
    pltpu.sync_copy(labels_hbm.at[pl.ds(base, _BPW)], lab_v)
    for g in range(_BPW // _L):
        lab = lab_v[pl.ds(g * _L, _L)]
        bidx = lax.iota(jnp.int32, _L) + (base + g * _L)
        flat = bidx * _C + lab
        row_v[pl.ds(g * _L, _L)] = lax.shift_right_logical(flat, 7)
    pltpu.async_copy(logits_hbm.at[row_v], rows_v, sem).wait()
    for g in range(_BPW // _L):
        lab = lab_v[pl.ds(g * _L, _L)]
        bidx = lax.iota(jnp.int32, _L) + (base + g * _L)
        lane = lax.bitwise_and(bidx * _C + lab, jnp.int32(_RW - 1))
        rowi = lax.iota(jnp.int32, _L) + g * _L
        val_v[pl.ds(g * _L, _L)] = plsc.load_gather(rows_v, [rowi, lane])
    pltpu.sync_copy(val_v, out_hbm.at[pl.ds(base, _BPW)])


def _sc_gather(logits, labels):
    k = functools.partial(
        pl.kernel,
        mesh=plsc.VectorSubcoreMesh(**_SC_MESH),
        compiler_params=pltpu.CompilerParams(needs_layout_passes=False),
        out_type=jax.ShapeDtypeStruct((_B,), jnp.float32),
        scratch_types=[
            pltpu.VMEM((_BPW,), jnp.int32),
            pltpu.VMEM((_BPW,), jnp.int32),
            pltpu.VMEM((_BPW, _RW), jnp.float32),
            pltpu.VMEM((_BPW,), jnp.float32),
            pltpu.SemaphoreType.DMA,
        ],
    )(_sc_gather_body)
    return k(logits.reshape(_B * _C // _RW, _RW), labels)


_LANES = 128
_PREFIX = 2048


def _slice_update(acc_ref, x_ref, vt, labm, c, chunk, s, extra_mask=None):
    xs = x_ref[:, s * _LANES:(s + 1) * _LANES]
    base = c * chunk + s * _LANES
    m_ge = xs >= vt
    m_gt = xs > vt
    if extra_mask is not None:
        m_ge = m_ge & extra_mask
        m_gt = m_gt & extra_mask
    mc = labm > base
    f_ge = jnp.where(m_ge, 1.0, 0.0)
    f_gt = jnp.where(m_gt, 1.0, 0.0)
    acc_ref[...] += jnp.where(mc, f_ge, f_gt)


def _prefix_body(v_ref, labm_ref, x_ref, out_ref, acc_ref):
    acc_ref[...] = jnp.zeros_like(acc_ref)
    vt = v_ref[...]
    labm = labm_ref[...]
    for s in range(_PREFIX // _LANES):
        _slice_update(acc_ref, x_ref, vt, labm, 0, _PREFIX, s)
    out_ref[...] = jnp.sum(acc_ref[...], axis=1, keepdims=True)


def _tc_prefix(logits, vb, labm):
    return pl.pallas_call(
        _prefix_body,
        grid=(1,),
        in_specs=[
            pl.BlockSpec((_B, _LANES), lambda c: (0, 0)),
            pl.BlockSpec((_B, _LANES), lambda c: (0, 0)),
            pl.BlockSpec((_B, _PREFIX), lambda c: (0, 0)),
        ],
        out_specs=pl.BlockSpec((_B, 1), lambda c: (0, 0)),
        out_shape=jax.ShapeDtypeStruct((_B, 1), jnp.float32),
        scratch_shapes=[pltpu.VMEM((_B, _LANES), jnp.float32)],
    )(vb, labm, logits)


_GRP = 10
_NGRP = _C // _L
assert _NGRP % _GRP == 0


def _sc_rows_body(flat_hbm, rows_hbm, vu_hbm, labml_hbm, n_hbm, out_hbm,
                  meta_v, vu_v, labml_v, buf, val_v, n_v):
    wid = lax.axis_index("s") * _NC + lax.axis_index("c")
    pltpu.sync_copy(n_hbm, n_v)
    n = jnp.max(n_v[...])
    trips = jnp.maximum(0, (n - wid + _NW - 1) // _NW)

    def one_slot(j, carry):
        u = wid + j * _NW
        pltpu.sync_copy(rows_hbm.at[u], meta_v)
        pltpu.sync_copy(vu_hbm.at[u], vu_v)
        pltpu.sync_copy(labml_hbm.at[u], labml_v)
        row = jnp.max(meta_v[...])
        start = row * _C
        pltpu.sync_copy(flat_hbm.at[pl.ds(start, _C)], buf)
        vu = vu_v[...]
        labml = labml_v[...]

        def step(i, acc):
            for uu in range(_GRP):
                g = i * _GRP + uu
                xs = buf[pl.ds(g * _L, _L)]
                m_ge = xs >= vu
                m_gt = xs > vu
                mc = labml > g * _L
                f_ge = jnp.where(m_ge, 1.0, 0.0)
                f_gt = jnp.where(m_gt, 1.0, 0.0)
                acc = acc + jnp.where(mc, f_ge, f_gt)
            return acc

        acc = lax.fori_loop(0, _NGRP // _GRP, step,
                            jnp.zeros((_L,), jnp.float32))
        count = jnp.sum(acc)
        val_v[...] = jnp.broadcast_to(count, (_L,))
        pltpu.sync_copy(val_v, out_hbm.at[u])
        return carry

    lax.fori_loop(0, trips, one_slot, 0)


def _sc_rows(flat_logits, rows_all, vu, labu, n):
    meta = jnp.broadcast_to(rows_all[:, None], (_B, _L)).astype(jnp.int32)
    vu16 = jnp.broadcast_to(vu[:, None], (_B, _L))
    labml = labu[:, None].astype(jnp.int32) - lax.broadcasted_iota(
        jnp.int32, (_B, _L), 1)
    n16 = jnp.broadcast_to(n.reshape(1), (_L,)).astype(jnp.int32)
    k = functools.partial(
        pl.kernel,
        mesh=plsc.VectorSubcoreMesh(**_SC_MESH),
        compiler_params=pltpu.CompilerParams(needs_layout_passes=False),
        out_type=jax.ShapeDtypeStruct((_B, _L), jnp.float32),
        scratch_types=[
            pltpu.VMEM((_L,), jnp.int32),
            pltpu.VMEM((_L,), jnp.float32),
            pltpu.VMEM((_L,), jnp.int32),
            pltpu.VMEM((_C,), jnp.float32),
            pltpu.VMEM((_L,), jnp.float32),
            pltpu.VMEM((_L,), jnp.int32),
        ],
    )(_sc_rows_body)
    return k(flat_logits, meta, vu16, labml, n16)


def kernel(logits, labels):
    labels = labels.astype(jnp.int32)
    v = _sc_gather(logits, labels)

    vb = jnp.broadcast_to(v.reshape(_B, 1), (_B, _LANES))
    labm = labels.reshape(_B, 1) - lax.broadcasted_iota(
        jnp.int32, (_B, _LANES), 1)

    prefix = _tc_prefix(logits, vb, labm).reshape(_B)
    undecided = prefix < float(_K)
    n = jnp.sum(undecided.astype(jnp.int32))
    (rows_all,) = jnp.nonzero(undecided, size=_B, fill_value=0)
    rows_all = rows_all.astype(jnp.int32)

    vu = v[rows_all]
    labu = labels[rows_all]
    counts = _sc_rows(logits.reshape(-1), rows_all, vu, labu, n)[:, 0]
    live = lax.broadcasted_iota(jnp.int32, (_B,), 0) < n
    correct = (counts < float(_K)) & live
    return jnp.sum(correct.astype(jnp.float32)) * (1.0 / _B)


# speedup vs baseline: 4.3972x; 3.6050x over previous
import functools

import jax
import jax.numpy as jnp
from jax import lax
from jax.experimental import pallas as pl
from jax.experimental.pallas import tpu as pltpu
from jax.experimental.pallas import tpu_sc as plsc

_B = 1024
_C = 100000
_K = 10

_NC = 2
_NS = 16
_NW = _NC * _NS
_L = 16
_BPW = _B // _NW

_SC_MESH = dict(core_axis_name="c", subcore_axis_name="s")
_SC_PARAMS = dict(
    compiler_params=pltpu.CompilerParams(needs_layout_passes=False))

_LANES = 128
_ALIGNED = (_C // _LANES) * _LANES
_TAILW = _C - _ALIGNED
_TAILP = 128
_WIN = 128
_COL_MAX = _ALIGNED - _WIN


def _sc_gather_body(logits_hbm, tail_hbm, labels_hbm, out_hbm,
                    lab_v, rows_v, val_v, sem):
    wid = lax.axis_index("s") * _NC + lax.axis_index("c")
    base = wid * _BPW
    pltpu.sync_copy(labels_hbm.at[pl.ds(base, _BPW)], lab_v)
    lane16 = lax.iota(jnp.int32, _L)
    copies = []
    for j in range(_BPW):
        labs = lab_v[pl.ds((j // _L) * _L, _L)]
        lab_j = jnp.sum(jnp.where(lane16 == (j % _L), labs, 0))
        col0_j = pl.multiple_of(
            jnp.minimum(lab_j & jnp.int32(~(_WIN - 1)), _COL_MAX), _WIN)
        row0_j = pl.multiple_of(base + (j // 8) * 8, 8)
        copies.append(pltpu.async_copy(
            logits_hbm.at[pl.ds(row0_j, 8), pl.ds(col0_j, _WIN)],
            rows_v.at[pl.ds(j * 8, 8), pl.ds(0, _WIN)], sem))
    for cp in copies:
        cp.wait()
    for j in range(_BPW):
        labs = lab_v[pl.ds((j // _L) * _L, _L)]
        lab_j = jnp.sum(jnp.where(lane16 == (j % _L), labs, 0))
        row0_j = pl.multiple_of(base + (j // 8) * 8, 8)

        @pl.when(lab_j >= _ALIGNED)
        def _fetch_tail():
            pltpu.sync_copy(tail_hbm.at[pl.ds(row0_j, 8)],
                            rows_v.at[pl.ds(j * 8, 8)])

    for g in range(_BPW // _L):
        lab = lab_v[pl.ds(g * _L, _L)]
        col0 = jnp.minimum(lab & jnp.int32(~(_WIN - 1)), _COL_MAX)
        is_tail = lab >= _ALIGNED
        coloff = jnp.where(is_tail, lab - _ALIGNED, lab - col0)
        slot = lane16 + g * _L
        vrow = slot * 8 + (slot & 7)
        val_v[pl.ds(g * _L, _L)] = plsc.load_gather(rows_v, [vrow, coloff])
    pltpu.sync_copy(val_v, out_hbm.at[pl.ds(base, _BPW)])


def _sc_gather(logits, tail_pad, labels):
    k = functools.partial(
        pl.kernel,
        mesh=plsc.VectorSubcoreMesh(**_SC_MESH),
        out_type=jax.ShapeDtypeStruct((_B,), jnp.float32),
        scratch_types=[
            pltpu.VMEM((_BPW,), jnp.int32),
            pltpu.VMEM((_BPW * 8, _TAILP), jnp.float32),
            pltpu.VMEM((_BPW,), jnp.float32),
            pltpu.SemaphoreType.DMA,
        ],
        **_SC_PARAMS,
    )(_sc_gather_body)
    return k(logits, tail_pad, labels)


_PREFIX = 2048
_PCHUNK = 512
_PSTEPS = _PREFIX // _PCHUNK


def _slice_update(acc_ref, xs, vt, labm, base, extra_mask=None):
    m_ge = xs >= vt
    m_gt = xs > vt
    if extra_mask is not None:
        m_ge = m_ge & extra_mask
        m_gt = m_gt & extra_mask
    mc = labm > base
    f_ge = jnp.where(m_ge, 1.0, 0.0)
    f_gt = jnp.where(m_gt, 1.0, 0.0)
    acc_ref[...] += jnp.where(mc, f_ge, f_gt)


def _prefix_body(v_ref, labm_ref, x_ref, xt_ref, pre_ref, tail_ref,
                 acc_ref, acct_ref):
    c = pl.program_id(0)

    @pl.when(c == 0)
    def _init():
        acc_ref[...] = jnp.zeros_like(acc_ref)

    vt = v_ref[...]
    labm = labm_ref[...]
    for s in range(_PCHUNK // _LANES):
        xs = x_ref[:, s * _LANES:(s + 1) * _LANES]
        _slice_update(acc_ref, xs, vt, labm, c * _PCHUNK + s * _LANES)

    @pl.when(c == _PSTEPS - 1)
    def _fini():
        pre_ref[...] = jnp.sum(acc_ref[...], axis=1, keepdims=True)
        acct_ref[...] = jnp.zeros_like(acct_ref)
        for s in range(_TAILP // _LANES):
            xs = xt_ref[:, s * _LANES:(s + 1) * _LANES]
            width = _TAILW - s * _LANES
            mask = (lax.broadcasted_iota(jnp.int32, (_B, _LANES), 1) < width
                    ) if width < _LANES else None
            _slice_update(acct_ref, xs, vt, labm, _ALIGNED + s * _LANES,
                          extra_mask=mask)
        tail_ref[...] = jnp.sum(acct_ref[...], axis=1, keepdims=True)


def _tc_prefix(logits, vb, labm):
    return pl.pallas_call(
        _prefix_body,
        grid=(_PSTEPS,),
        in_specs=[
            pl.BlockSpec((_B, _LANES), lambda c: (0, 0)),
            pl.BlockSpec((_B, _LANES), lambda c: (0, 0)),
            pl.BlockSpec((_B, _PCHUNK), lambda c: (0, c)),
            pl.BlockSpec((_B, _TAILP), lambda c: (0, _ALIGNED // _TAILP)),
        ],
        out_specs=[
            pl.BlockSpec((_B, 1), lambda c: (0, 0)),
            pl.BlockSpec((_B, 1), lambda c: (0, 0)),
        ],
        out_shape=[
            jax.ShapeDtypeStruct((_B, 1), jnp.float32),
            jax.ShapeDtypeStruct((_B, 1), jnp.float32),
        ],
        scratch_shapes=[
            pltpu.VMEM((_B, _LANES), jnp.float32),
            pltpu.VMEM((_B, _LANES), jnp.float32),
        ],
    )(vb, labm, logits, logits)


_CW = 9088
_NCHUNK = _ALIGNED // _CW
assert _NCHUNK * _CW == _ALIGNED
_GRP = 8
_GPC = _CW // _L
assert _GPC % _GRP == 0


def _sc_rows_body(logits_hbm, rows_hbm, vu_hbm, labml_hbm, n_hbm, out_hbm,
                  meta_v, vu_v, labml_v, buf, val_v, n_v):
    wid = lax.axis_index("s") * _NC + lax.axis_index("c")
    pltpu.sync_copy(n_hbm, n_v)
    n = jnp.max(n_v[...])
    ntasks = n * _NCHUNK
    trips = jnp.maximum(0, (ntasks - wid + _NW - 1) // _NW)
    lane16 = lax.iota(jnp.int32, _L)

    def one_task(j, carry):
        t = wid + j * _NW
        u = t // _NCHUNK
        c = t - u * _NCHUNK
        pltpu.sync_copy(rows_hbm.at[u], meta_v)
        pltpu.sync_copy(vu_hbm.at[u], vu_v)
        pltpu.sync_copy(labml_hbm.at[u], labml_v)
        row = jnp.max(meta_v[...])
        row0 = pl.multiple_of((row >> 3) << 3, 8)
        roff = row - row0
        ws = pl.multiple_of(c * _CW, _LANES)
        pltpu.sync_copy(logits_hbm.at[pl.ds(row0, 8), pl.ds(ws, _CW)], buf)
        vu = vu_v[...]
        labml = labml_v[...]
        roff16 = jnp.broadcast_to(roff, (_L,))

        def step(i, acc):
            for uu in range(_GRP):
                g = i * _GRP + uu
                xs = plsc.load_gather(buf, [roff16, lane16 + g * _L])
                m_ge = xs >= vu
                m_gt = xs > vu
                mc = labml > ws + g * _L
                f_ge = jnp.where(m_ge, 1.0, 0.0)
                f_gt = jnp.where(m_gt, 1.0, 0.0)
                acc = acc + jnp.where(mc, f_ge, f_gt)
            return acc

        acc = lax.fori_loop(0, _GPC // _GRP, step,
                            jnp.zeros((_L,), jnp.float32))
        count = jnp.sum(acc)
        val_v[...] = jnp.broadcast_to(count, (_L,))
        pltpu.sync_copy(val_v, out_hbm.at[u, c])
        return carry

    lax.fori_loop(0, trips, one_task, 0)


def _sc_rows(logits, rows_all, vu, labu, n):
    meta = jnp.broadcast_to(rows_all[:, None], (_B, _L)).astype(jnp.int32)
    vu16 = jnp.broadcast_to(vu[:, None], (_B, _L))
    labml = labu[:, None].astype(jnp.int32) - lax.broadcasted_iota(
        jnp.int32, (_B, _L), 1)
    n16 = jnp.broadcast_to(n.reshape(1), (_L,)).astype(jnp.int32)
    k = functools.partial(
        pl.kernel,
        mesh=plsc.VectorSubcoreMesh(**_SC_MESH),
        out_type=jax.ShapeDtypeStruct((_B, _NCHUNK, _L), jnp.float32),
        scratch_types=[
            pltpu.VMEM((_L,), jnp.int32),
            pltpu.VMEM((_L,), jnp.float32),
            pltpu.VMEM((_L,), jnp.int32),
            pltpu.VMEM((8, _CW), jnp.float32),
            pltpu.VMEM((_L,), jnp.float32),
            pltpu.VMEM((_L,), jnp.int32),
        ],
        **_SC_PARAMS,
    )(_sc_rows_body)
    return k(logits, meta, vu16, labml, n16)


def kernel(logits, labels):
    labels = labels.astype(jnp.int32)
    tail_pad = jnp.pad(logits[:, _ALIGNED:], ((0, 0), (0, _TAILP - _TAILW)))
    v = _sc_gather(logits, tail_pad, labels)

    vb = jnp.broadcast_to(v.reshape(_B, 1), (_B, _LANES))
    labm = labels.reshape(_B, 1) - lax.broadcasted_iota(
        jnp.int32, (_B, _LANES), 1)

    pre, tailcnt = _tc_prefix(logits, vb, labm)
    pre = pre.reshape(_B)
    tailcnt = tailcnt.reshape(_B)
    undecided = (pre + tailcnt) < float(_K)
    n = jnp.sum(undecided.astype(jnp.int32))
    (rows_all,) = jnp.nonzero(undecided, size=_B, fill_value=0)
    rows_all = rows_all.astype(jnp.int32)

    vu = v[rows_all]
    labu = labels[rows_all]
    chunk_counts = _sc_rows(logits, rows_all, vu, labu, n)[:, :, 0]
    counts = jnp.sum(chunk_counts, axis=1) + tailcnt[rows_all]
    live = lax.broadcasted_iota(jnp.int32, (_B,), 0) < n
    correct = (counts < float(_K)) & live
    return jnp.sum(correct.astype(jnp.float32)) * (1.0 / _B)


# speedup vs baseline: 11.4582x; 2.6058x over previous
import functools

import jax
import jax.numpy as jnp
from jax import lax
from jax.experimental import pallas as pl
from jax.experimental.pallas import tpu as pltpu
from jax.experimental.pallas import tpu_sc as plsc

_B = 1024
_C = 100000
_K = 10

_NC = 2
_NS = 16
_NW = _NC * _NS
_L = 16
_BPW = _B // _NW
_LANES = 128

_SC_MESH = dict(core_axis_name="c", subcore_axis_name="s")
_SC_PARAMS = dict(
    compiler_params=pltpu.CompilerParams(needs_layout_passes=False))


def _sc_gather_body(xt_hbm, labels_hbm, out_hbm, lab_v, rows_v, val_v, sem):
    wid = lax.axis_index("s") * _NC + lax.axis_index("c")
    base = wid * _BPW
    pltpu.sync_copy(labels_hbm.at[pl.ds(base, _BPW)], lab_v)
    lane16 = lax.iota(jnp.int32, _L)
    copies = []
    for j in range(_BPW):
        labs = lab_v[pl.ds((j // _L) * _L, _L)]
        lab_j = jnp.sum(jnp.where(lane16 == (j % _L), labs, 0))
        lab0_j = pl.multiple_of((lab_j >> 3) << 3, 8)
        col0_j = pl.multiple_of(((base + j) >> 7) << 7, _LANES)
        copies.append(pltpu.async_copy(
            xt_hbm.at[pl.ds(lab0_j, 8), pl.ds(col0_j, _LANES)],
            rows_v.at[pl.ds(j * 8, 8)], sem))
    for cp in copies:
        cp.wait()
    for g in range(_BPW // _L):
        lab = lab_v[pl.ds(g * _L, _L)]
        slot = lane16 + g * _L
        vrow = slot * 8 + (lab & 7)
        coloff = (base + slot) & (_LANES - 1)
        val_v[pl.ds(g * _L, _L)] = plsc.load_gather(rows_v, [vrow, coloff])
    pltpu.sync_copy(val_v, out_hbm.at[pl.ds(base, _BPW)])


def _sc_gather(xt, labels):
    k = functools.partial(
        pl.kernel,
        mesh=plsc.VectorSubcoreMesh(**_SC_MESH),
        out_type=jax.ShapeDtypeStruct((_B,), jnp.float32),
        scratch_types=[
            pltpu.VMEM((_BPW,), jnp.int32),
            pltpu.VMEM((_BPW * 8, _LANES), jnp.float32),
            pltpu.VMEM((_BPW,), jnp.float32),
            pltpu.SemaphoreType.DMA,
        ],
        **_SC_PARAMS,
    )(_sc_gather_body)
    return k(xt, labels)


_SLC = 128
_CHUNK = 2048
_NCH = -(-_C // _CHUNK)
_TAIL = _C - (_NCH - 1) * _CHUNK
_TAIL_FULL = _TAIL // _SLC
_TAIL_REM = _TAIL - _TAIL_FULL * _SLC


def _count_body(v_ref, labm_ref, x_ref, out_ref, acc_ref):
    c = pl.program_id(0)

    @pl.when(c == 0)
    def _init():
        acc_ref[...] = jnp.zeros_like(acc_ref)

    vt = v_ref[...]
    labm = labm_ref[...]

    def slice_update(s, rows):
        xs = x_ref[s * _SLC:s * _SLC + rows, :]
        base = c * _CHUNK + s * _SLC
        vs = vt if rows == _SLC else vt[:rows, :]
        lm = labm if rows == _SLC else labm[:rows, :]
        m_ge = xs >= vs
        m_gt = xs > vs
        mc = lm > base
        f_ge = jnp.where(m_ge, 1.0, 0.0)
        f_gt = jnp.where(m_gt, 1.0, 0.0)
        if rows == _SLC:
            acc_ref[...] += jnp.where(mc, f_ge, f_gt)
        else:
            acc_ref[:rows, :] += jnp.where(mc, f_ge, f_gt)

    @pl.when(c < _NCH - 1)
    def _full():
        for s in range(_CHUNK // _SLC):
            slice_update(s, _SLC)

    @pl.when(c == _NCH - 1)
    def _tail_and_fini():
        for s in range(_TAIL_FULL):
            slice_update(s, _SLC)
        slice_update(_TAIL_FULL, _TAIL_REM)
        counts = jnp.sum(acc_ref[...], axis=0, keepdims=True)
        correct = (counts < float(_K)).astype(jnp.float32)
        total = jnp.sum(correct) * (1.0 / _B)
        out_ref[...] = jnp.broadcast_to(total, (1, 1))


def _tc_count(xt, vb, labm):
    return pl.pallas_call(
        _count_body,
        grid=(_NCH,),
        in_specs=[
            pl.BlockSpec((_SLC, _B), lambda c: (0, 0)),
            pl.BlockSpec((_SLC, _B), lambda c: (0, 0)),
            pl.BlockSpec((_CHUNK, _B), lambda c: (c, 0)),
        ],
        out_specs=pl.BlockSpec((1, 1), lambda c: (0, 0)),
        out_shape=jax.ShapeDtypeStruct((1, 1), jnp.float32),
        scratch_shapes=[pltpu.VMEM((_SLC, _B), jnp.float32)],
    )(vb, labm, xt).reshape(())


def kernel(logits, labels):
    labels = labels.astype(jnp.int32)
    xt = logits.T
    v = _sc_gather(xt, labels)
    vb = jnp.broadcast_to(v[None, :], (_SLC, _B))
    labm = labels[None, :] - lax.broadcasted_iota(jnp.int32, (_SLC, _B), 0)
    return _tc_count(xt, vb, labm)


# speedup vs baseline: 12.1652x; 1.0617x over previous
import functools

import jax
import jax.numpy as jnp
from jax import lax
from jax.experimental import pallas as pl
from jax.experimental.pallas import tpu as pltpu
from jax.experimental.pallas import tpu_sc as plsc

_B = 1024
_C = 100000
_K = 10

_NC = 2
_NS = 16
_NW = _NC * _NS
_L = 16
_BPW = _B // _NW
_LANES = 128

_SC_MESH = dict(core_axis_name="c", subcore_axis_name="s")
_SC_PARAMS = dict(
    compiler_params=pltpu.CompilerParams(needs_layout_passes=False))


def _sc_gather_body(xt_hbm, labels_hbm, out_hbm, lab_v, rows_v, val_v, sem):
    wid = lax.axis_index("s") * _NC + lax.axis_index("c")
    base = wid * _BPW
    pltpu.sync_copy(labels_hbm.at[pl.ds(base, _BPW)], lab_v)
    lane16 = lax.iota(jnp.int32, _L)
    copies = []
    for j in range(_BPW):
        labs = lab_v[pl.ds((j // _L) * _L, _L)]
        lab_j = jnp.sum(jnp.where(lane16 == (j % _L), labs, 0))
        lab0_j = pl.multiple_of((lab_j >> 3) << 3, 8)
        col0_j = pl.multiple_of(((base + j) >> 7) << 7, _LANES)
        copies.append(pltpu.async_copy(
            xt_hbm.at[pl.ds(lab0_j, 8), pl.ds(col0_j, _LANES)],
            rows_v.at[pl.ds(j * 8, 8)], sem))
    for cp in copies:
        cp.wait()
    for g in range(_BPW // _L):
        lab = lab_v[pl.ds(g * _L, _L)]
        slot = lane16 + g * _L
        vrow = slot * 8 + (lab & 7)
        coloff = (base + slot) & (_LANES - 1)
        val_v[pl.ds(g * _L, _L)] = plsc.load_gather(rows_v, [vrow, coloff])
    pltpu.sync_copy(val_v, out_hbm.at[pl.ds(base, _BPW)])


def _sc_gather(xt, labels):
    k = functools.partial(
        pl.kernel,
        mesh=plsc.VectorSubcoreMesh(**_SC_MESH),
        out_type=jax.ShapeDtypeStruct((_B,), jnp.float32),
        scratch_types=[
            pltpu.VMEM((_BPW,), jnp.int32),
            pltpu.VMEM((_BPW * 8, _LANES), jnp.float32),
            pltpu.VMEM((_BPW,), jnp.float32),
            pltpu.SemaphoreType.DMA,
        ],
        **_SC_PARAMS,
    )(_sc_gather_body)
    return k(xt, labels)


_SLC = 128
_CHUNK = 3072
_NCH = -(-_C // _CHUNK)
_TAIL = _C - (_NCH - 1) * _CHUNK
_TAIL_FULL = _TAIL // _SLC
_TAIL_REM = _TAIL - _TAIL_FULL * _SLC


def _count_body(v_ref, vp_ref, labm_ref, x_ref, out_ref, acc_ref):
    c = pl.program_id(0)

    @pl.when(c == 0)
    def _init():
        acc_ref[...] = jnp.zeros_like(acc_ref)

    vt = v_ref[...]
    vp = vp_ref[...]
    labm = labm_ref[...]

    def slice_update(s, rows):
        xs = x_ref[s * _SLC:s * _SLC + rows, :]
        base = c * _CHUNK + s * _SLC
        vs = vt if rows == _SLC else vt[:rows, :]
        vps = vp if rows == _SLC else vp[:rows, :]
        lm = labm if rows == _SLC else labm[:rows, :]
        mc = lm > base
        thr = jnp.where(mc, vps, vs)
        m = xs > thr
        if rows == _SLC:
            acc_ref[...] += jnp.where(m, 1.0, 0.0)
        else:
            acc_ref[:rows, :] += jnp.where(m, 1.0, 0.0)

    @pl.when(c < _NCH - 1)
    def _full():
        for s in range(_CHUNK // _SLC):
            slice_update(s, _SLC)

    @pl.when(c == _NCH - 1)
    def _tail_and_fini():
        for s in range(_TAIL_FULL):
            slice_update(s, _SLC)
        slice_update(_TAIL_FULL, _TAIL_REM)
        counts = jnp.sum(acc_ref[...], axis=0, keepdims=True)
        correct = (counts < float(_K)).astype(jnp.float32)
        total = jnp.sum(correct) * (1.0 / _B)
        out_ref[...] = jnp.broadcast_to(total, (1, 1))


def _tc_count(xt, vb, vpb, labm):
    return pl.pallas_call(
        _count_body,
        grid=(_NCH,),
        in_specs=[
            pl.BlockSpec((_SLC, _B), lambda c: (0, 0)),
            pl.BlockSpec((_SLC, _B), lambda c: (0, 0)),
            pl.BlockSpec((_SLC, _B), lambda c: (0, 0)),
            pl.BlockSpec((_CHUNK, _B), lambda c: (c, 0)),
        ],
        out_specs=pl.BlockSpec((1, 1), lambda c: (0, 0)),
        out_shape=jax.ShapeDtypeStruct((1, 1), jnp.float32),
        scratch_shapes=[pltpu.VMEM((_SLC, _B), jnp.float32)],
    )(vb, vpb, labm, xt).reshape(())


def _pred32(v):
    b = lax.bitcast_convert_type(v, jnp.int32)
    bp = jnp.where(v > 0, b - 1, b + 1)
    bp = jnp.where(v == 0, jnp.int32(-0x7FFFFFFF), bp)
    return lax.bitcast_convert_type(bp, jnp.float32)


def kernel(logits, labels):
    labels = labels.astype(jnp.int32)
    xt = logits.T
    v = _sc_gather(xt, labels)
    vb = jnp.broadcast_to(v[None, :], (_SLC, _B))
    vpb = jnp.broadcast_to(_pred32(v)[None, :], (_SLC, _B))
    labm = labels[None, :] - lax.broadcasted_iota(jnp.int32, (_SLC, _B), 0)
    return _tc_count(xt, vb, vpb, labm)


# speedup vs baseline: 12.3326x; 1.0138x over previous
import functools

import jax
import jax.numpy as jnp
from jax import lax
from jax.experimental import pallas as pl
from jax.experimental.pallas import tpu as pltpu
from jax.experimental.pallas import tpu_sc as plsc

_B = 1024
_C = 100000
_K = 10

_NC = 2
_NS = 16
_NW = _NC * _NS
_L = 16
_BPW = _B // _NW
_LANES = 128

_SC_MESH = dict(core_axis_name="c", subcore_axis_name="s")
_SC_PARAMS = dict(
    compiler_params=pltpu.CompilerParams(needs_layout_passes=False))


def _sc_gather_body(xt_hbm, labels_hbm, out_hbm, lab_v, rows_v, val_v, sem):
    wid = lax.axis_index("s") * _NC + lax.axis_index("c")
    base = wid * _BPW
    pltpu.sync_copy(labels_hbm.at[pl.ds(base, _BPW)], lab_v)
    lane16 = lax.iota(jnp.int32, _L)
    copies = []
    for j in range(_BPW):
        labs = lab_v[pl.ds((j // _L) * _L, _L)]
        lab_j = jnp.sum(jnp.where(lane16 == (j % _L), labs, 0))
        lab0_j = pl.multiple_of((lab_j >> 3) << 3, 8)
        col0_j = pl.multiple_of(((base + j) >> 7) << 7, _LANES)
        copies.append(pltpu.async_copy(
            xt_hbm.at[pl.ds(lab0_j, 8), pl.ds(col0_j, _LANES)],
            rows_v.at[pl.ds(j * 8, 8)], sem))
    for cp in copies:
        cp.wait()
    for g in range(_BPW // _L):
        lab = lab_v[pl.ds(g * _L, _L)]
        slot = lane16 + g * _L
        vrow = slot * 8 + (lab & 7)
        coloff = (base + slot) & (_LANES - 1)
        val_v[pl.ds(g * _L, _L)] = plsc.load_gather(rows_v, [vrow, coloff])
    pltpu.sync_copy(val_v, out_hbm.at[pl.ds(base, _BPW)])


def _sc_gather(xt, labels):
    k = functools.partial(
        pl.kernel,
        mesh=plsc.VectorSubcoreMesh(**_SC_MESH),
        out_type=jax.ShapeDtypeStruct((_B,), jnp.float32),
        scratch_types=[
            pltpu.VMEM((_BPW,), jnp.int32),
            pltpu.VMEM((_BPW * 8, _LANES), jnp.float32),
            pltpu.VMEM((_BPW,), jnp.float32),
            pltpu.SemaphoreType.DMA,
        ],
        **_SC_PARAMS,
    )(_sc_gather_body)
    return k(xt, labels)


_SLC = 128
_CHUNK = 4096
_NCH = -(-_C // _CHUNK)
_TAIL = _C - (_NCH - 1) * _CHUNK
_TAIL_FULL = _TAIL // _SLC
_TAIL_REM = _TAIL - _TAIL_FULL * _SLC


def _count_body(v_ref, vp_ref, labm_ref, x_ref, out_ref, acc_ref):
    c = pl.program_id(0)

    @pl.when(c == 0)
    def _init():
        acc_ref[...] = jnp.zeros_like(acc_ref)

    vt = v_ref[...]
    vp = vp_ref[...]
    labm = labm_ref[...]

    def slice_update(s, rows):
        xs = x_ref[s * _SLC:s * _SLC + rows, :]
        base = c * _CHUNK + s * _SLC
        vs = vt if rows == _SLC else vt[:rows, :]
        vps = vp if rows == _SLC else vp[:rows, :]
        lm = labm if rows == _SLC else labm[:rows, :]
        mc = lm > base
        thr = jnp.where(mc, vps, vs)
        m = xs > thr
        if rows == _SLC:
            acc_ref[...] += jnp.where(m, 1.0, 0.0)
        else:
            acc_ref[:rows, :] += jnp.where(m, 1.0, 0.0)

    @pl.when(c < _NCH - 1)
    def _full():
        for s in range(_CHUNK // _SLC):
            slice_update(s, _SLC)

    @pl.when(c == _NCH - 1)
    def _tail_and_fini():
        for s in range(_TAIL_FULL):
            slice_update(s, _SLC)
        slice_update(_TAIL_FULL, _TAIL_REM)
        counts = jnp.sum(acc_ref[...], axis=0, keepdims=True)
        correct = (counts < float(_K)).astype(jnp.float32)
        total = jnp.sum(correct) * (1.0 / _B)
        out_ref[...] = jnp.broadcast_to(total, (1, 1))


def _tc_count(xt, vb, vpb, labm):
    return pl.pallas_call(
        _count_body,
        grid=(_NCH,),
        in_specs=[
            pl.BlockSpec((_SLC, _B), lambda c: (0, 0)),
            pl.BlockSpec((_SLC, _B), lambda c: (0, 0)),
            pl.BlockSpec((_SLC, _B), lambda c: (0, 0)),
            pl.BlockSpec((_CHUNK, _B), lambda c: (c, 0)),
        ],
        out_specs=pl.BlockSpec((1, 1), lambda c: (0, 0)),
        out_shape=jax.ShapeDtypeStruct((1, 1), jnp.float32),
        scratch_shapes=[pltpu.VMEM((_SLC, _B), jnp.float32)],
    )(vb, vpb, labm, xt).reshape(())


def _pred32(v):
    b = lax.bitcast_convert_type(v, jnp.int32)
    bp = jnp.where(v > 0, b - 1, b + 1)
    bp = jnp.where(v == 0, jnp.int32(-0x7FFFFFFF), bp)
    return lax.bitcast_convert_type(bp, jnp.float32)


def kernel(logits, labels):
    labels = labels.astype(jnp.int32)
    xt = logits.T
    v = _sc_gather(xt, labels)
    vb = jnp.broadcast_to(v[None, :], (_SLC, _B))
    vpb = jnp.broadcast_to(_pred32(v)[None, :], (_SLC, _B))
    labm = labels[None, :] - lax.broadcasted_iota(jnp.int32, (_SLC, _B), 0)
    return _tc_count(xt, vb, vpb, labm)
